# baseline (speedup 1.0000x reference)
import functools

import jax
import jax.numpy as jnp
from jax import lax
from jax.experimental import pallas as pl
from jax.experimental.pallas import tpu as pltpu
from jax.experimental.pallas import tpu_sc as plsc


def _sc_build(s_len, n0, n1, d0, d1, nc, ns):
    nw = nc * ns
    j_per_w = n1 // nw
    bc_rows = n0 // 2

    mesh = plsc.VectorSubcoreMesh(core_axis_name="c", subcore_axis_name="s")

    @functools.partial(
        pl.kernel,
        out_type=jax.ShapeDtypeStruct((s_len, d0 + d1), jnp.float32),
        mesh=mesh,
        scratch_types=[
            pltpu.VMEM((bc_rows, d1), jnp.float32),
            pltpu.VMEM((bc_rows, d1), jnp.float32),
            pltpu.VMEM((j_per_w, bc_rows), jnp.int32),
            pltpu.SemaphoreType.DMA,
            pltpu.SemaphoreType.DMA,
            pltpu.SemaphoreType.DMA,
            pltpu.SemaphoreType.DMA,
            pltpu.SemaphoreType.DMA,
        ],
    )
    def body(x1_hbm, x2_hbm, out_hbm, bc0, bc1, idx_v, sx, sg0, sg1, sw0, sw1):
        wid = lax.axis_index("s") * nc + lax.axis_index("c")
        bufs = (bc0, bc1)
        gsems = (sg0, sg1)
        wsems = (sw0, sw1)

        xw = []
        for t in range(j_per_w):
            j = wid * j_per_w + t
            xw.append(
                pltpu.async_copy(
                    x1_hbm, out_hbm.at[pl.ds(j * n0, n0), pl.ds(0, d0)], sx
                )
            )

        for t in range(j_per_w):
            j = wid * j_per_w + t
            jvec = jnp.full((16,), j, jnp.int32)
            for q in range(bc_rows // 16):
                idx_v[t, pl.ds(q * 16, 16)] = jvec

        gathers = [None, None]
        writes = [[], []]
        gathers[0] = pltpu.async_copy(x2_hbm.at[idx_v.at[0]], bufs[0], gsems[0])
        for t in range(j_per_w):
            b = t % 2
            gathers[b].wait()
            j = wid * j_per_w + t
            base = j * n0
            for h in range(2):
                writes[b].append(
                    pltpu.async_copy(
                        bufs[b],
                        out_hbm.at[pl.ds(base + h * bc_rows, bc_rows), pl.ds(d0, d1)],
                        wsems[b],
                    )
                )
            if t + 1 < j_per_w:
                nb = (t + 1) % 2
                for w in writes[nb]:
                    w.wait()
                writes[nb] = []
                gathers[nb] = pltpu.async_copy(
                    x2_hbm.at[idx_v.at[t + 1]], bufs[nb], gsems[nb]
                )

        for ws in writes:
            for w in ws:
                w.wait()
        for w in xw:
            w.wait()

    return body


def kernel(x, x1, x2):
    s_len = x.shape[1]
    n0, d0 = x1.shape
    n1, d1 = x2.shape
    info = plsc.get_sparse_core_info()
    build = _sc_build(s_len, n0, n1, d0, d1, info.num_cores, info.num_subcores)
    out = build(x1, x2)
    return out.astype(x.dtype)[None, :, :]


# speedup vs baseline: 14.4669x; 14.4669x over previous
import functools

import jax
import jax.numpy as jnp
from jax import lax
from jax.experimental import pallas as pl
from jax.experimental.pallas import tpu as pltpu
from jax.experimental.pallas import tpu_sc as plsc


def _sc_build(s_len, n0, n1, d0, d1, nc, ns):
    nw = nc * ns
    j_per_w = n1 // nw
    bc_rows = n0 // 2

    mesh = plsc.VectorSubcoreMesh(core_axis_name="c", subcore_axis_name="s")

    @functools.partial(
        pl.kernel,
        out_type=jax.ShapeDtypeStruct((s_len, d0 + d1), jnp.float32),
        mesh=mesh,
        scratch_types=[
            pltpu.VMEM_SHARED((n0, d0), jnp.float32),
            pltpu.VMEM((bc_rows, d1), jnp.float32),
            pltpu.VMEM((bc_rows, d1), jnp.float32),
            pltpu.VMEM((j_per_w, bc_rows), jnp.int32),
            pltpu.SemaphoreType.DMA,
            pltpu.SemaphoreType.DMA,
            pltpu.SemaphoreType.DMA,
            pltpu.SemaphoreType.DMA,
            pltpu.SemaphoreType.DMA,
        ],
    )
    def body(x1_hbm, x2_hbm, out_hbm, x1_sh, bc0, bc1, idx_v, sx, sg0, sg1, sw0, sw1):
        wid = lax.axis_index("s") * nc + lax.axis_index("c")
        bufs = (bc0, bc1)
        gsems = (sg0, sg1)
        wsems = (sw0, sw1)

        @pl.when(lax.axis_index("s") == 0)
        def _():
            pltpu.sync_copy(x1_hbm, x1_sh)

        plsc.subcore_barrier()

        xw = []
        for t in range(j_per_w):
            j = wid * j_per_w + t
            xw.append(
                pltpu.async_copy(
                    x1_sh, out_hbm.at[pl.ds(j * n0, n0), pl.ds(0, d0)], sx
                )
            )

        for t in range(j_per_w):
            j = wid * j_per_w + t
            jvec = jnp.full((16,), j, jnp.int32)
            for q in range(bc_rows // 16):
                idx_v[t, pl.ds(q * 16, 16)] = jvec

        gathers = [None, None]
        writes = [[], []]
        gathers[0] = pltpu.async_copy(x2_hbm.at[idx_v.at[0]], bufs[0], gsems[0])
        for t in range(j_per_w):
            b = t % 2
            gathers[b].wait()
            j = wid * j_per_w + t
            base = j * n0
            for h in range(2):
                writes[b].append(
                    pltpu.async_copy(
                        bufs[b],
                        out_hbm.at[pl.ds(base + h * bc_rows, bc_rows), pl.ds(d0, d1)],
                        wsems[b],
                    )
                )
            if t + 1 < j_per_w:
                nb = (t + 1) % 2
                for w in writes[nb]:
                    w.wait()
                writes[nb] = []
                gathers[nb] = pltpu.async_copy(
                    x2_hbm.at[idx_v.at[t + 1]], bufs[nb], gsems[nb]
                )

        for ws in writes:
            for w in ws:
                w.wait()
        for w in xw:
            w.wait()

    return body


def kernel(x, x1, x2):
    s_len = x.shape[1]
    n0, d0 = x1.shape
    n1, d1 = x2.shape
    info = plsc.get_sparse_core_info()
    build = _sc_build(s_len, n0, n1, d0, d1, info.num_cores, info.num_subcores)
    out = build(x1, x2)
    return out.astype(x.dtype)[None, :, :]
